# Initial kernel scaffold; baseline (speedup 1.0000x reference)
import jax
import jax.numpy as jnp
from jax.experimental import pallas as pl


def kernel(node_fts, adj, lengths, enc_W, enc_b, m1_W, m2_W, msg_b, o1_W, o2_W, o_b, dec_W, dec_b):
    raise NotImplementedError("write your pallas kernel here")


# speedup vs baseline: 2.3491x; 2.3491x over previous
import functools

import jax
import jax.numpy as jnp
from jax.experimental import pallas as pl
from jax.experimental.pallas import tpu as pltpu

_B, _N, _T = 16, 128, 17
_D_IN, _H = 128, 128
_BIG = 100000.0


def _mpnn_kernel(lengths_ref, node_ref, adj_ref, encW_ref, encb_ref,
                 Wall_ref, msgb_ref, o2_ref, ob_ref, decW_ref, decb_ref,
                 out_ref):
    b = pl.program_id(0)
    x = jnp.dot(node_ref[0], encW_ref[...],
                preferred_element_type=jnp.float32) + encb_ref[...]
    adj = adj_ref[0]
    hasnb = jnp.max(adj, axis=1, keepdims=True) > 0.0
    nsteps = jnp.maximum(lengths_ref[b] - 1, 1)

    def step(_, h):
        z = jnp.concatenate([x, h], axis=1)
        r = jnp.dot(z, Wall_ref[...], preferred_element_type=jnp.float32)
        msg1 = r[:, :_H]
        msg2 = r[:, _H:2 * _H]
        zo1 = r[:, 2 * _H:]
        M = jnp.full((_N, _H), -_BIG, dtype=jnp.float32)
        for s in range(_N):
            col = adj[:, s:s + 1]
            row = msg2[s:s + 1, :]
            M = jnp.maximum(M, jnp.where(col > 0.0, row, -_BIG))
        agg = jnp.where(hasnb,
                        jnp.maximum(msg1 + msgb_ref[...] + M, 0.0),
                        -_BIG)
        h_new = jnp.maximum(
            zo1 + jnp.dot(agg, o2_ref[...],
                          preferred_element_type=jnp.float32) + ob_ref[...],
            0.0)
        return h_new

    h = jax.lax.fori_loop(0, nsteps, step, jnp.zeros((_N, _H), jnp.float32),
                          unroll=False)
    z = jnp.concatenate([x, h], axis=1)
    out_ref[0] = (jnp.dot(z, decW_ref[...],
                          preferred_element_type=jnp.float32) + decb_ref[0, 0])


@jax.jit
def kernel(node_fts, adj, lengths, enc_W, enc_b, m1_W, m2_W, msg_b,
           o1_W, o2_W, o_b, dec_W, dec_b):
    W_all = jnp.concatenate([m1_W, m2_W, o1_W], axis=1)
    grid = (_B,)
    full = lambda shape: pl.BlockSpec(shape, lambda i: (0,) * len(shape))
    out = pl.pallas_call(
        _mpnn_kernel,
        grid=grid,
        in_specs=[
            pl.BlockSpec(memory_space=pltpu.SMEM),
            pl.BlockSpec((1, _N, _D_IN), lambda i: (i, 0, 0)),
            pl.BlockSpec((1, _N, _N), lambda i: (i, 0, 0)),
            full((_D_IN, _H)),
            full((1, _H)),
            full((2 * _H, 3 * _H)),
            full((1, _H)),
            full((_H, _H)),
            full((1, _H)),
            full((2 * _H, 1)),
            full((1, 1)),
        ],
        out_specs=pl.BlockSpec((1, _N, 1), lambda i: (i, 0, 0)),
        out_shape=jax.ShapeDtypeStruct((_B, _N, 1), jnp.float32),
        compiler_params=pltpu.CompilerParams(
            dimension_semantics=("arbitrary",)),
    )(lengths, node_fts, adj, enc_W, enc_b.reshape(1, _H), W_all,
      msg_b.reshape(1, _H), o2_W, o_b.reshape(1, _H), dec_W,
      dec_b.reshape(1, 1))
    return out[:, :, 0]


# speedup vs baseline: 4.3990x; 1.8726x over previous
import functools

import jax
import jax.numpy as jnp
from jax.experimental import pallas as pl
from jax.experimental.pallas import tpu as pltpu

_B, _N, _T = 16, 128, 17
_D_IN, _H = 128, 128
_BIG = 100000.0


def _mpnn_kernel(lengths_ref, node_ref, adj_ref, encW_ref, encb_ref,
                 Wall_ref, msgb_ref, o2_ref, ob_ref, decW_ref, decb_ref,
                 out_ref, bias_ref):
    b = pl.program_id(0)
    x = jnp.dot(node_ref[0], encW_ref[...],
                preferred_element_type=jnp.float32) + encb_ref[...]
    adj = adj_ref[0]
    hasnb = jnp.max(adj, axis=1, keepdims=True) > 0.0
    nsteps = jnp.maximum(lengths_ref[b] - 1, 1)

    for s in range(_N):
        bias_ref[s] = jnp.where(
            jax.lax.broadcast_in_dim(adj[:, s:s + 1], (_N, _H), (0, 1)) > 0.0,
            jnp.inf, -jnp.inf)

    def step(_, h):
        z = jnp.concatenate([x, h], axis=1)
        r = jnp.dot(z, Wall_ref[...], preferred_element_type=jnp.float32)
        msg1 = r[:, :_H]
        msg2 = r[:, _H:2 * _H]
        zo1 = r[:, 2 * _H:3 * _H]
        M = jnp.full((_N, _H), -_BIG, dtype=jnp.float32)
        for s in range(_N):
            row = msg2[s:s + 1, :]
            M = jnp.maximum(M, jnp.minimum(row, bias_ref[s]))
        agg = jnp.where(hasnb,
                        jnp.maximum((msg1 + M) + msgb_ref[...], 0.0),
                        -_BIG)
        h_new = jnp.maximum(
            zo1 + jnp.dot(agg, o2_ref[...],
                          preferred_element_type=jnp.float32) + ob_ref[...],
            0.0)
        return h_new

    h = jax.lax.fori_loop(0, nsteps, step, jnp.zeros((_N, _H), jnp.float32),
                          unroll=False)
    z = jnp.concatenate([x, h], axis=1)
    out_ref[0] = (jnp.dot(z, decW_ref[...],
                          preferred_element_type=jnp.float32) + decb_ref[0, 0])


@jax.jit
def kernel(node_fts, adj, lengths, enc_W, enc_b, m1_W, m2_W, msg_b,
           o1_W, o2_W, o_b, dec_W, dec_b):
    W_all = jnp.concatenate(
        [m1_W, m2_W, o1_W, jnp.zeros_like(o1_W)], axis=1)
    grid = (_B,)
    full = lambda shape: pl.BlockSpec(shape, lambda i: (0,) * len(shape))
    out = pl.pallas_call(
        _mpnn_kernel,
        grid=grid,
        in_specs=[
            pl.BlockSpec(memory_space=pltpu.SMEM),
            pl.BlockSpec((1, _N, _D_IN), lambda i: (i, 0, 0)),
            pl.BlockSpec((1, _N, _N), lambda i: (i, 0, 0)),
            full((_D_IN, _H)),
            full((1, _H)),
            full((2 * _H, 4 * _H)),
            full((1, _H)),
            full((_H, _H)),
            full((1, _H)),
            full((2 * _H, 1)),
            full((1, 1)),
        ],
        out_specs=pl.BlockSpec((1, _N, 1), lambda i: (i, 0, 0)),
        out_shape=jax.ShapeDtypeStruct((_B, _N, 1), jnp.float32),
        scratch_shapes=[pltpu.VMEM((_N, _N, _H), jnp.float32)],
        compiler_params=pltpu.CompilerParams(
            dimension_semantics=("arbitrary",)),
    )(lengths, node_fts, adj, enc_W, enc_b.reshape(1, _H), W_all,
      msg_b.reshape(1, _H), o2_W, o_b.reshape(1, _H), dec_W,
      dec_b.reshape(1, 1))
    return out[:, :, 0]


# speedup vs baseline: 4.7316x; 1.0756x over previous
import functools

import jax
import jax.numpy as jnp
from jax.experimental import pallas as pl
from jax.experimental.pallas import tpu as pltpu

_B, _N, _T = 16, 128, 17
_D_IN, _H = 128, 128
_BIG = 100000.0


def _mpnn_kernel(lengths_ref, node_ref, adj_ref, encW_ref, encb_ref,
                 Wall_ref, msgb_ref, o2_ref, ob_ref, decW_ref, decb_ref,
                 out_ref, bias_ref):
    b = pl.program_id(0)
    x = jnp.dot(node_ref[0], encW_ref[...],
                preferred_element_type=jnp.float32) + encb_ref[...]
    adj = adj_ref[0]
    hasnb = jnp.max(adj, axis=1, keepdims=True) > 0.0
    nsteps = jnp.maximum(lengths_ref[b] - 1, 1)

    def compute_r(h):
        z = jnp.concatenate([x, h], axis=1)
        return jnp.dot(z, Wall_ref[...], preferred_element_type=jnp.float32)

    def finish(r, M):
        agg = jnp.where(hasnb,
                        jnp.maximum((r[:, :_H] + M) + msgb_ref[...], 0.0),
                        -_BIG)
        return jnp.maximum(
            r[:, 2 * _H:3 * _H] +
            jnp.dot(agg, o2_ref[...],
                    preferred_element_type=jnp.float32) + ob_ref[...],
            0.0)

    r = compute_r(jnp.zeros((_N, _H), jnp.float32))
    msg2 = r[:, _H:2 * _H]
    M = jnp.full((_N, _H), -_BIG, dtype=jnp.float32)
    for s in range(_N):
        bl = (jax.lax.broadcast_in_dim(adj[:, s:s + 1], (_N, _H), (0, 1))
              * 2e30 - 1e30)
        bias_ref[s] = bl
        M = jnp.maximum(M, jnp.minimum(msg2[s:s + 1, :], bl))
    h = finish(r, M)

    def step(_, h):
        r = compute_r(h)
        msg2 = r[:, _H:2 * _H]
        M = jnp.full((_N, _H), -_BIG, dtype=jnp.float32)
        for s in range(_N):
            M = jnp.maximum(M, jnp.minimum(msg2[s:s + 1, :], bias_ref[s]))
        return finish(r, M)

    h = jax.lax.fori_loop(1, nsteps, step, h, unroll=False)
    z = jnp.concatenate([x, h], axis=1)
    out_ref[0] = (jnp.dot(z, decW_ref[...],
                          preferred_element_type=jnp.float32) + decb_ref[0, 0])


@jax.jit
def kernel(node_fts, adj, lengths, enc_W, enc_b, m1_W, m2_W, msg_b,
           o1_W, o2_W, o_b, dec_W, dec_b):
    W_all = jnp.concatenate(
        [m1_W, m2_W, o1_W, jnp.zeros_like(o1_W)], axis=1)
    grid = (_B,)
    full = lambda shape: pl.BlockSpec(shape, lambda i: (0,) * len(shape))
    out = pl.pallas_call(
        _mpnn_kernel,
        grid=grid,
        in_specs=[
            pl.BlockSpec(memory_space=pltpu.SMEM),
            pl.BlockSpec((1, _N, _D_IN), lambda i: (i, 0, 0)),
            pl.BlockSpec((1, _N, _N), lambda i: (i, 0, 0)),
            full((_D_IN, _H)),
            full((1, _H)),
            full((2 * _H, 4 * _H)),
            full((1, _H)),
            full((_H, _H)),
            full((1, _H)),
            full((2 * _H, 1)),
            full((1, 1)),
        ],
        out_specs=pl.BlockSpec((1, _N, 1), lambda i: (i, 0, 0)),
        out_shape=jax.ShapeDtypeStruct((_B, _N, 1), jnp.float32),
        scratch_shapes=[pltpu.VMEM((_N, _N, _H), jnp.float32)],
        compiler_params=pltpu.CompilerParams(
            dimension_semantics=("arbitrary",)),
    )(lengths, node_fts, adj, enc_W, enc_b.reshape(1, _H), W_all,
      msg_b.reshape(1, _H), o2_W, o_b.reshape(1, _H), dec_W,
      dec_b.reshape(1, 1))
    return out[:, :, 0]
